# baseline (speedup 1.0000x reference)
import functools

import jax
import jax.numpy as jnp
from jax.experimental import pallas as pl
from jax.experimental.pallas import tpu as pltpu

_NUM_CODES = 256


def _vq_body(x_ref, w_ref, q_ref, e_ref):
    x = x_ref[0]
    w = w_ref[...]
    xw = jax.lax.dot_general(x, w, (((0,), (1,)), ((), ())),
                             preferred_element_type=jnp.float32)
    x2 = jnp.sum(x * x, axis=0)
    w2 = jnp.sum(w * w, axis=1)
    d = x2[:, None] + w2[None, :] - 2.0 * xw
    idx = jnp.argmin(d, axis=1)
    k_iota = jax.lax.broadcasted_iota(jnp.int32, d.shape, 1)
    e = (k_iota == idx[:, None]).astype(jnp.float32)
    e_ref[...] = e
    q_ref[0] = jax.lax.dot_general(w, e, (((0,), (1,)), ((), ())),
                                   preferred_element_type=jnp.float32)


@functools.partial(jax.jit, static_argnames=("interpret",))
def kernel(inputs, W, interpret=False):
    B, C, H, Wd = inputs.shape
    P = H * Wd
    K = W.shape[0]
    x3 = inputs.reshape(B, C, P)
    q3, e = pl.pallas_call(
        _vq_body,
        grid=(B,),
        in_specs=[
            pl.BlockSpec((1, C, P), lambda b: (b, 0, 0)),
            pl.BlockSpec((K, C), lambda b: (0, 0)),
        ],
        out_specs=[
            pl.BlockSpec((1, C, P), lambda b: (b, 0, 0)),
            pl.BlockSpec((P, K), lambda b: (b, 0)),
        ],
        out_shape=[
            jax.ShapeDtypeStruct((B, C, P), jnp.float32),
            jax.ShapeDtypeStruct((B * P, K), jnp.float32),
        ],
        interpret=interpret,
    )(x3, W)
    return q3.reshape(B, C, H, Wd), e


# speedup vs baseline: 1.5279x; 1.5279x over previous
import functools

import jax
import jax.numpy as jnp
from jax.experimental import pallas as pl
from jax.experimental.pallas import tpu as pltpu


def _vq_body(x_ref, w_ref, wt_ref, q_ref, e_ref):
    x = x_ref[0]
    w = w_ref[...]
    wt = wt_ref[...]
    K, P = w.shape[0], x.shape[1]
    xw = jax.lax.dot_general(w, x, (((1,), (0,)), ((), ())),
                             preferred_element_type=jnp.float32)
    x2 = jnp.sum(x * x, axis=0)
    w2 = jnp.sum(w * w, axis=1)
    d = (x2[None, :] + w2[:, None]) - 2.0 * xw
    m = jnp.min(d, axis=0)
    kk = jax.lax.broadcasted_iota(jnp.int32, d.shape, 0)
    idx = jnp.min(jnp.where(d == m[None, :], kk, K), axis=0)
    et = (kk == idx[None, :]).astype(jnp.float32)
    q_ref[0] = jax.lax.dot_general(wt, et, (((1,), (0,)), ((), ())),
                                   preferred_element_type=jnp.float32)
    idx_col = jnp.transpose(idx.reshape(1, P))
    p_iota = jax.lax.broadcasted_iota(jnp.int32, (P, K), 1)
    e_ref[...] = (p_iota == idx_col).astype(jnp.float32)


@functools.partial(jax.jit, static_argnames=("interpret",))
def kernel(inputs, W, interpret=False):
    B, C, H, Wd = inputs.shape
    P = H * Wd
    K = W.shape[0]
    x3 = inputs.reshape(B, C, P)
    q3, e = pl.pallas_call(
        _vq_body,
        grid=(B,),
        in_specs=[
            pl.BlockSpec((1, C, P), lambda b: (b, 0, 0)),
            pl.BlockSpec((K, C), lambda b: (0, 0)),
            pl.BlockSpec((C, K), lambda b: (0, 0)),
        ],
        out_specs=[
            pl.BlockSpec((1, C, P), lambda b: (b, 0, 0)),
            pl.BlockSpec((P, K), lambda b: (b, 0)),
        ],
        out_shape=[
            jax.ShapeDtypeStruct((B, C, P), jnp.float32),
            jax.ShapeDtypeStruct((B * P, K), jnp.float32),
        ],
        interpret=interpret,
    )(x3, W, W.T)
    return q3.reshape(B, C, H, Wd), e


# speedup vs baseline: 1.7382x; 1.1376x over previous
import functools

import jax
import jax.numpy as jnp
from jax.experimental import pallas as pl
from jax.experimental.pallas import tpu as pltpu


def _vq_body(x_ref, w_ref, wt_ref, q_ref, e_ref):
    nb = x_ref.shape[0]
    w = w_ref[...]
    wt = wt_ref[...]
    K = w.shape[0]
    P = x_ref.shape[2]
    w2 = jnp.sum(w * w, axis=1)
    for i in range(nb):
        x = x_ref[i]
        xw = jax.lax.dot_general(w, x, (((1,), (0,)), ((), ())),
                                 preferred_element_type=jnp.float32)
        x2 = jnp.sum(x * x, axis=0)
        d = (x2[None, :] + w2[:, None]) - 2.0 * xw
        m = jnp.min(d, axis=0)
        kk = jax.lax.broadcasted_iota(jnp.int32, d.shape, 0)
        idx = jnp.min(jnp.where(d == m[None, :], kk, K), axis=0)
        et = (kk == idx[None, :]).astype(jnp.float32)
        q_ref[i] = jax.lax.dot_general(wt, et, (((1,), (0,)), ((), ())),
                                       preferred_element_type=jnp.float32)
        idx_col = jnp.transpose(idx.reshape(1, P))
        p_iota = jax.lax.broadcasted_iota(jnp.int32, (P, K), 1)
        e_ref[pl.ds(i * P, P), :] = (p_iota == idx_col).astype(jnp.float32)


@functools.partial(jax.jit, static_argnames=("interpret",))
def kernel(inputs, W, interpret=False):
    B, C, H, Wd = inputs.shape
    P = H * Wd
    K = W.shape[0]
    x3 = inputs.reshape(B, C, P)
    NB = 2
    q3, e = pl.pallas_call(
        _vq_body,
        grid=(B // NB,),
        in_specs=[
            pl.BlockSpec((NB, C, P), lambda b: (b, 0, 0)),
            pl.BlockSpec((K, C), lambda b: (0, 0)),
            pl.BlockSpec((C, K), lambda b: (0, 0)),
        ],
        out_specs=[
            pl.BlockSpec((NB, C, P), lambda b: (b, 0, 0)),
            pl.BlockSpec((NB * P, K), lambda b: (b, 0)),
        ],
        out_shape=[
            jax.ShapeDtypeStruct((B, C, P), jnp.float32),
            jax.ShapeDtypeStruct((B * P, K), jnp.float32),
        ],
        interpret=interpret,
    )(x3, W, W.T)
    return q3.reshape(B, C, H, Wd), e


# speedup vs baseline: 1.8357x; 1.0561x over previous
import functools

import jax
import jax.numpy as jnp
from jax.experimental import pallas as pl
from jax.experimental.pallas import tpu as pltpu


def _vq_body(x_ref, w_ref, wt_ref, q_ref, e_ref):
    nb = x_ref.shape[0]
    w = w_ref[...]
    wt = wt_ref[...]
    K = w.shape[0]
    P = x_ref.shape[2]
    w2 = jnp.sum(w * w, axis=1)
    for i in range(nb):
        x = x_ref[i]
        xw = jax.lax.dot_general(w, x, (((1,), (0,)), ((), ())),
                                 preferred_element_type=jnp.float32)
        x2 = jnp.sum(x * x, axis=0)
        d = (x2[None, :] + w2[:, None]) - 2.0 * xw
        m = jnp.min(d, axis=0)
        kk = jax.lax.broadcasted_iota(jnp.int32, d.shape, 0)
        idx = jnp.min(jnp.where(d == m[None, :], kk, K), axis=0)
        et = (kk == idx[None, :]).astype(jnp.float32)
        q_ref[i] = jax.lax.dot_general(wt, et, (((1,), (0,)), ((), ())),
                                       preferred_element_type=jnp.float32)
        idx_col = jnp.transpose(idx.reshape(1, P))
        p_iota = jax.lax.broadcasted_iota(jnp.int32, (P, K), 1)
        e_ref[pl.ds(i * P, P), :] = (p_iota == idx_col).astype(jnp.float32)


@functools.partial(jax.jit, static_argnames=("interpret",))
def kernel(inputs, W, interpret=False):
    B, C, H, Wd = inputs.shape
    P = H * Wd
    K = W.shape[0]
    x3 = inputs.reshape(B, C, P)
    NB = 4
    q3, e = pl.pallas_call(
        _vq_body,
        grid=(B // NB,),
        in_specs=[
            pl.BlockSpec((NB, C, P), lambda b: (b, 0, 0)),
            pl.BlockSpec((K, C), lambda b: (0, 0)),
            pl.BlockSpec((C, K), lambda b: (0, 0)),
        ],
        out_specs=[
            pl.BlockSpec((NB, C, P), lambda b: (b, 0, 0)),
            pl.BlockSpec((NB * P, K), lambda b: (b, 0)),
        ],
        out_shape=[
            jax.ShapeDtypeStruct((B, C, P), jnp.float32),
            jax.ShapeDtypeStruct((B * P, K), jnp.float32),
        ],
        interpret=interpret,
    )(x3, W, W.T)
    return q3.reshape(B, C, H, Wd), e
